# baseline (speedup 1.0000x reference)
import jax
import jax.numpy as jnp
from jax.experimental import pallas as pl
from jax.experimental.pallas import tpu as pltpu

IDIM = 64
K = 1000
KPAD = 1024
TNUM = 10
NITER = 10
CA = 385.0 / NITER
CB = 2.0 * 55.0 / NITER
BLK = 2048


def _vq_loss_kernel(x_ref, xso_ref, valid_ref, e_ref, w_ref, b_ref, kb_ref,
                    out_ref, ea_ref, ew_ref, accv_ref):
    i = pl.program_id(0)
    nsteps = pl.num_programs(0)

    @pl.when(i == 0)
    def _init():
        e = e_ref[...]
        norm2 = jnp.sum(e * e, axis=1, keepdims=True)
        inv = jnp.where(norm2 > 0.0, 1.0 / jnp.sqrt(norm2), 0.0)
        ea_ref[:, 0:IDIM] = e * inv
        ea_ref[:, IDIM:IDIM + 1] = kb_ref[...]
        ea_ref[:, IDIM + 1:] = jnp.zeros((KPAD, IDIM - 1), jnp.float32)
        ew_ref[:, 0:IDIM] = jax.lax.dot(e, w_ref[...],
                                        preferred_element_type=jnp.float32)
        ew_ref[:, IDIM:2 * IDIM] = jnp.ones((KPAD, IDIM), jnp.float32)
        accv_ref[...] = jnp.zeros_like(accv_ref)

    x = x_ref[0]
    s = jax.lax.dot_general(x, ea_ref[...], (((1,), (1,)), ((), ())),
                            preferred_element_type=jnp.float32)
    m = jnp.max(s, axis=1, keepdims=True)
    onehot = (s == m).astype(jnp.float32)
    p2 = jax.lax.dot(onehot, ew_ref[...],
                     preferred_element_type=jnp.float32)
    p = p2[:, 0:IDIM] / p2[:, IDIM:2 * IDIM] + b_ref[...]

    xo = xso_ref[0]
    t2 = (xo[:, 0:128] + xo[:, 128:256] + xo[:, 256:384]
          + xo[:, 384:512] + xo[:, 512:640])
    t = (t2[:, 0:IDIM] + t2[:, IDIM:2 * IDIM]) * (1.0 / TNUM)

    v = valid_ref[0]
    accv_ref[...] += (CA * (p * p) - CB * (p * t) + t * t) * v

    @pl.when(i == nsteps - 1)
    def _fin():
        out_ref[...] = jnp.reshape(jnp.sum(accv_ref[...]), (1, 1))


def _run(xs_pad_in, xs_pad_out, ilens, embed_weight, W_inf, b_inf,
         interpret=False):
    B, T, _ = xs_pad_in.shape
    N = B * T
    tb = T // BLK
    xso = xs_pad_out.reshape(B, T, TNUM * IDIM)
    xa = jnp.concatenate(
        [xs_pad_in, jnp.ones((B, T, 1), jnp.float32),
         jnp.zeros((B, T, IDIM - 1), jnp.float32)], axis=-1)
    valid = (jnp.arange(T, dtype=jnp.int32)[None, :, None]
             < ilens[:, None, None].astype(jnp.int32)).astype(jnp.float32)
    epad = jnp.zeros((KPAD, IDIM), jnp.float32).at[:K, :].set(embed_weight)
    kb = jnp.where(jnp.arange(KPAD)[:, None] < K, 0.0, -1e30
                   ).astype(jnp.float32)
    b2 = b_inf.reshape(1, IDIM)

    grid = (N // BLK,)
    out = pl.pallas_call(
        _vq_loss_kernel,
        grid=grid,
        in_specs=[
            pl.BlockSpec((1, BLK, 2 * IDIM), lambda i: (i // tb, i % tb, 0)),
            pl.BlockSpec((1, BLK, TNUM * IDIM),
                         lambda i: (i // tb, i % tb, 0)),
            pl.BlockSpec((1, BLK, 1), lambda i: (i // tb, i % tb, 0)),
            pl.BlockSpec((KPAD, IDIM), lambda i: (0, 0)),
            pl.BlockSpec((IDIM, IDIM), lambda i: (0, 0)),
            pl.BlockSpec((1, IDIM), lambda i: (0, 0)),
            pl.BlockSpec((KPAD, 1), lambda i: (0, 0)),
        ],
        out_specs=pl.BlockSpec((1, 1), lambda i: (0, 0)),
        out_shape=jax.ShapeDtypeStruct((1, 1), jnp.float32),
        scratch_shapes=[
            pltpu.VMEM((KPAD, 2 * IDIM), jnp.float32),
            pltpu.VMEM((KPAD, 2 * IDIM), jnp.float32),
            pltpu.VMEM((BLK, IDIM), jnp.float32),
        ],
        interpret=interpret,
    )(xa, xso, valid, epad, W_inf, b2, kb)
    return out.reshape(())


def kernel(xs_pad_in, xs_pad_out, ilens, ys_pad, embed_weight, W_inf, b_inf):
    return _run(xs_pad_in, xs_pad_out, ilens, embed_weight, W_inf, b_inf)


# speedup vs baseline: 1.1025x; 1.1025x over previous
import jax
import jax.numpy as jnp
from jax.experimental import pallas as pl
from jax.experimental.pallas import tpu as pltpu

IDIM = 64
K = 1000
KPAD = 1024
TNUM = 10
NITER = 10
CA = 385.0 / NITER
CB = 2.0 * 55.0 / NITER
BLK = 2048


def _vq_loss_kernel(x_ref, xso_ref, valid_ref, e_ref, w_ref, b_ref, kb_ref,
                    out_ref, ea_ref, ew_ref, accv_ref):
    i = pl.program_id(0)
    nsteps = pl.num_programs(0)

    @pl.when(i == 0)
    def _init():
        e = e_ref[...]
        norm2 = jnp.sum(e * e, axis=1, keepdims=True)
        inv = jnp.where(norm2 > 0.0, 1.0 / jnp.sqrt(norm2), 0.0)
        ea_ref[...] = e * inv
        ew_ref[:, 0:IDIM] = jax.lax.dot(e, w_ref[...],
                                        preferred_element_type=jnp.float32)
        ew_ref[:, IDIM:2 * IDIM] = jnp.ones((KPAD, IDIM), jnp.float32)
        accv_ref[...] = jnp.zeros_like(accv_ref)

    x = x_ref[0]
    s = jax.lax.dot_general(x, ea_ref[...], (((1,), (1,)), ((), ())),
                            preferred_element_type=jnp.float32)
    s = s + kb_ref[...]
    m = jnp.max(s, axis=1, keepdims=True)
    onehot = (s == m).astype(jnp.float32)
    p2 = jax.lax.dot(onehot, ew_ref[...],
                     preferred_element_type=jnp.float32)
    p = p2[:, 0:IDIM] / p2[:, IDIM:2 * IDIM] + b_ref[...]

    xo = xso_ref[0]
    t2 = (xo[:, 0:128] + xo[:, 128:256] + xo[:, 256:384]
          + xo[:, 384:512] + xo[:, 512:640])
    t = (t2[:, 0:IDIM] + t2[:, IDIM:2 * IDIM]) * (1.0 / TNUM)

    v = valid_ref[0]
    accv_ref[...] += (CA * (p * p) - CB * (p * t) + t * t) * v

    @pl.when(i == nsteps - 1)
    def _fin():
        out_ref[...] = jnp.reshape(jnp.sum(accv_ref[...]), (1, 1))


def _run(xs_pad_in, xs_pad_out, ilens, embed_weight, W_inf, b_inf,
         interpret=False):
    B, T, _ = xs_pad_in.shape
    N = B * T
    tb = T // BLK
    xso = xs_pad_out.reshape(B, T, TNUM * IDIM)
    valid = (jnp.arange(T, dtype=jnp.int32)[None, :, None]
             < ilens[:, None, None].astype(jnp.int32)).astype(jnp.float32)
    epad = jnp.zeros((KPAD, IDIM), jnp.float32).at[:K, :].set(embed_weight)
    kb = jnp.where(jnp.arange(KPAD)[None, :] < K, 0.0, -1e30
                   ).astype(jnp.float32)
    b2 = b_inf.reshape(1, IDIM)

    grid = (N // BLK,)
    out = pl.pallas_call(
        _vq_loss_kernel,
        grid=grid,
        in_specs=[
            pl.BlockSpec((1, BLK, IDIM), lambda i: (i // tb, i % tb, 0)),
            pl.BlockSpec((1, BLK, TNUM * IDIM),
                         lambda i: (i // tb, i % tb, 0)),
            pl.BlockSpec((1, BLK, 1), lambda i: (i // tb, i % tb, 0)),
            pl.BlockSpec((KPAD, IDIM), lambda i: (0, 0)),
            pl.BlockSpec((IDIM, IDIM), lambda i: (0, 0)),
            pl.BlockSpec((1, IDIM), lambda i: (0, 0)),
            pl.BlockSpec((1, KPAD), lambda i: (0, 0)),
        ],
        out_specs=pl.BlockSpec((1, 1), lambda i: (0, 0)),
        out_shape=jax.ShapeDtypeStruct((1, 1), jnp.float32),
        scratch_shapes=[
            pltpu.VMEM((KPAD, IDIM), jnp.float32),
            pltpu.VMEM((KPAD, 2 * IDIM), jnp.float32),
            pltpu.VMEM((BLK, IDIM), jnp.float32),
        ],
        interpret=interpret,
    )(xs_pad_in, xso, valid, epad, W_inf, b2, kb)
    return out.reshape(())


def kernel(xs_pad_in, xs_pad_out, ilens, ys_pad, embed_weight, W_inf, b_inf):
    return _run(xs_pad_in, xs_pad_out, ilens, embed_weight, W_inf, b_inf)


# speedup vs baseline: 1.1785x; 1.0689x over previous
import jax
import jax.numpy as jnp
from jax.experimental import pallas as pl
from jax.experimental.pallas import tpu as pltpu

IDIM = 64
K = 1000
KPAD = 1024
TNUM = 10
NITER = 10
CA = 385.0 / NITER
CB = 2.0 * 55.0 / NITER
HBLK = 1024


def _vq_loss_kernel(x_ref, xo1_ref, xo2_ref, valid_ref, e_ref, w_ref, b_ref,
                    kb_ref, out_ref, ea_ref, ew_ref, accv_ref):
    i = pl.program_id(0)
    nsteps = pl.num_programs(0)

    @pl.when(i == 0)
    def _init():
        e = e_ref[...]
        norm2 = jnp.sum(e * e, axis=1, keepdims=True)
        inv = jnp.where(norm2 > 0.0, 1.0 / jnp.sqrt(norm2), 0.0)
        ea_ref[...] = e * inv
        ew_ref[:, 0:IDIM] = jax.lax.dot(e, w_ref[...],
                                        preferred_element_type=jnp.float32)
        ew_ref[:, IDIM:2 * IDIM] = jnp.ones((KPAD, IDIM), jnp.float32)
        accv_ref[...] = jnp.zeros_like(accv_ref)

    for h, xo_ref in ((0, xo1_ref), (1, xo2_ref)):
        x = x_ref[0, h * HBLK:(h + 1) * HBLK, :]
        s = jax.lax.dot_general(x, ea_ref[...], (((1,), (1,)), ((), ())),
                                preferred_element_type=jnp.float32)
        s = s + kb_ref[...]
        m = jnp.max(s, axis=1, keepdims=True)
        onehot = (s == m).astype(jnp.float32)
        p2 = jax.lax.dot(onehot, ew_ref[...],
                         preferred_element_type=jnp.float32)
        p = p2[:, 0:IDIM] / p2[:, IDIM:2 * IDIM] + b_ref[...]

        xo = xo_ref[0]
        t2 = (xo[:, 0:128] + xo[:, 128:256] + xo[:, 256:384]
              + xo[:, 384:512] + xo[:, 512:640])
        t = (t2[:, 0:IDIM] + t2[:, IDIM:2 * IDIM]) * (1.0 / TNUM)

        v = valid_ref[0, h * HBLK:(h + 1) * HBLK, :]
        accv_ref[...] += (CA * (p * p) - CB * (p * t) + t * t) * v

    @pl.when(i == nsteps - 1)
    def _fin():
        out_ref[...] = jnp.reshape(jnp.sum(accv_ref[...]), (1, 1))


def _run(xs_pad_in, xs_pad_out, ilens, embed_weight, W_inf, b_inf,
         interpret=False):
    B, T, _ = xs_pad_in.shape
    xso = xs_pad_out.reshape(B, T, TNUM * IDIM)
    valid = (jnp.arange(T, dtype=jnp.int32)[None, :, None]
             < ilens[:, None, None].astype(jnp.int32)).astype(jnp.float32)
    epad = jnp.zeros((KPAD, IDIM), jnp.float32).at[:K, :].set(embed_weight)
    kb = jnp.where(jnp.arange(KPAD)[None, :] < K, 0.0, -1e30
                   ).astype(jnp.float32)
    b2 = b_inf.reshape(1, IDIM)

    grid = (B,)
    out = pl.pallas_call(
        _vq_loss_kernel,
        grid=grid,
        in_specs=[
            pl.BlockSpec((1, T, IDIM), lambda i: (i, 0, 0)),
            pl.BlockSpec((1, HBLK, TNUM * IDIM), lambda i: (i, 0, 0)),
            pl.BlockSpec((1, HBLK, TNUM * IDIM), lambda i: (i, 1, 0)),
            pl.BlockSpec((1, T, 1), lambda i: (i, 0, 0)),
            pl.BlockSpec((KPAD, IDIM), lambda i: (0, 0)),
            pl.BlockSpec((IDIM, IDIM), lambda i: (0, 0)),
            pl.BlockSpec((1, IDIM), lambda i: (0, 0)),
            pl.BlockSpec((1, KPAD), lambda i: (0, 0)),
        ],
        out_specs=pl.BlockSpec((1, 1), lambda i: (0, 0)),
        out_shape=jax.ShapeDtypeStruct((1, 1), jnp.float32),
        scratch_shapes=[
            pltpu.VMEM((KPAD, IDIM), jnp.float32),
            pltpu.VMEM((KPAD, 2 * IDIM), jnp.float32),
            pltpu.VMEM((HBLK, IDIM), jnp.float32),
        ],
        interpret=interpret,
    )(xs_pad_in, xso, xso, valid, epad, W_inf, b2, kb)
    return out.reshape(())


def kernel(xs_pad_in, xs_pad_out, ilens, ys_pad, embed_weight, W_inf, b_inf):
    return _run(xs_pad_in, xs_pad_out, ilens, embed_weight, W_inf, b_inf)


# speedup vs baseline: 1.2258x; 1.0401x over previous
import jax
import jax.numpy as jnp
from jax.experimental import pallas as pl
from jax.experimental.pallas import tpu as pltpu

IDIM = 64
K = 1000
KPAD = 1024
TNUM = 10
NITER = 10
CA = 385.0 / NITER
CB = 2.0 * 55.0 / NITER
HBLK = 512


def _vq_loss_kernel(x_ref, xo1_ref, xo2_ref, xo3_ref, xo4_ref, valid_ref,
                    e_ref, w_ref, b_ref, kb_ref, out_ref, ea_ref, ew_ref,
                    accv_ref):
    i = pl.program_id(0)
    nsteps = pl.num_programs(0)

    @pl.when(i == 0)
    def _init():
        e = e_ref[...]
        norm2 = jnp.sum(e * e, axis=1, keepdims=True)
        inv = jnp.where(norm2 > 0.0, 1.0 / jnp.sqrt(norm2), 0.0)
        ea_ref[...] = e * inv
        ew_ref[:, 0:IDIM] = jax.lax.dot(e, w_ref[...],
                                        preferred_element_type=jnp.float32)
        ew_ref[:, IDIM:2 * IDIM] = jnp.ones((KPAD, IDIM), jnp.float32)
        accv_ref[...] = jnp.zeros_like(accv_ref)

    for h, xo_ref in ((0, xo1_ref), (1, xo2_ref), (2, xo3_ref), (3, xo4_ref)):
        x = x_ref[0, h * HBLK:(h + 1) * HBLK, :]
        s = jax.lax.dot_general(x, ea_ref[...], (((1,), (1,)), ((), ())),
                                preferred_element_type=jnp.float32)
        s = s + kb_ref[...]
        m = jnp.max(s, axis=1, keepdims=True)
        onehot = (s == m).astype(jnp.float32)
        p2 = jax.lax.dot(onehot, ew_ref[...],
                         preferred_element_type=jnp.float32)
        p = p2[:, 0:IDIM] / p2[:, IDIM:2 * IDIM] + b_ref[...]

        xo = xo_ref[0]
        t2 = (xo[:, 0:128] + xo[:, 128:256] + xo[:, 256:384]
              + xo[:, 384:512] + xo[:, 512:640])
        t = (t2[:, 0:IDIM] + t2[:, IDIM:2 * IDIM]) * (1.0 / TNUM)

        v = valid_ref[0, h * HBLK:(h + 1) * HBLK, :]
        accv_ref[...] += (CA * (p * p) - CB * (p * t) + t * t) * v

    @pl.when(i == nsteps - 1)
    def _fin():
        out_ref[...] = jnp.reshape(jnp.sum(accv_ref[...]), (1, 1))


def _run(xs_pad_in, xs_pad_out, ilens, embed_weight, W_inf, b_inf,
         interpret=False):
    B, T, _ = xs_pad_in.shape
    xso = xs_pad_out.reshape(B, T, TNUM * IDIM)
    valid = (jnp.arange(T, dtype=jnp.int32)[None, :, None]
             < ilens[:, None, None].astype(jnp.int32)).astype(jnp.float32)
    epad = jnp.zeros((KPAD, IDIM), jnp.float32).at[:K, :].set(embed_weight)
    kb = jnp.where(jnp.arange(KPAD)[None, :] < K, 0.0, -1e30
                   ).astype(jnp.float32)
    b2 = b_inf.reshape(1, IDIM)

    grid = (B,)
    out = pl.pallas_call(
        _vq_loss_kernel,
        grid=grid,
        in_specs=[
            pl.BlockSpec((1, T, IDIM), lambda i: (i, 0, 0)),
            pl.BlockSpec((1, HBLK, TNUM * IDIM), lambda i: (i, 0, 0)),
            pl.BlockSpec((1, HBLK, TNUM * IDIM), lambda i: (i, 1, 0)),
            pl.BlockSpec((1, HBLK, TNUM * IDIM), lambda i: (i, 2, 0)),
            pl.BlockSpec((1, HBLK, TNUM * IDIM), lambda i: (i, 3, 0)),
            pl.BlockSpec((1, T, 1), lambda i: (i, 0, 0)),
            pl.BlockSpec((KPAD, IDIM), lambda i: (0, 0)),
            pl.BlockSpec((IDIM, IDIM), lambda i: (0, 0)),
            pl.BlockSpec((1, IDIM), lambda i: (0, 0)),
            pl.BlockSpec((1, KPAD), lambda i: (0, 0)),
        ],
        out_specs=pl.BlockSpec((1, 1), lambda i: (0, 0)),
        out_shape=jax.ShapeDtypeStruct((1, 1), jnp.float32),
        scratch_shapes=[
            pltpu.VMEM((KPAD, IDIM), jnp.float32),
            pltpu.VMEM((KPAD, 2 * IDIM), jnp.float32),
            pltpu.VMEM((HBLK, IDIM), jnp.float32),
        ],
        interpret=interpret,
    )(xs_pad_in, xso, xso, xso, xso, valid, epad, W_inf, b2, kb)
    return out.reshape(())


def kernel(xs_pad_in, xs_pad_out, ilens, ys_pad, embed_weight, W_inf, b_inf):
    return _run(xs_pad_in, xs_pad_out, ilens, embed_weight, W_inf, b_inf)


# speedup vs baseline: 1.2387x; 1.0105x over previous
import jax
import jax.numpy as jnp
from jax.experimental import pallas as pl
from jax.experimental.pallas import tpu as pltpu

IDIM = 64
K = 1000
KPAD = 1024
TNUM = 10
NITER = 10
CA = 385.0 / NITER
CB = 2.0 * 55.0 / NITER
HBLK = 256


def _vq_loss_kernel(x_ref, xo1_ref, xo2_ref, xo3_ref, xo4_ref, xo5_ref,
                    xo6_ref, xo7_ref, xo8_ref, valid_ref,
                    e_ref, w_ref, b_ref, kb_ref, out_ref, ea_ref, ew_ref,
                    accv_ref):
    i = pl.program_id(0)
    nsteps = pl.num_programs(0)

    @pl.when(i == 0)
    def _init():
        e = e_ref[...]
        norm2 = jnp.sum(e * e, axis=1, keepdims=True)
        inv = jnp.where(norm2 > 0.0, 1.0 / jnp.sqrt(norm2), 0.0)
        ea_ref[...] = e * inv
        ew_ref[:, 0:IDIM] = jax.lax.dot(e, w_ref[...],
                                        preferred_element_type=jnp.float32)
        ew_ref[:, IDIM:2 * IDIM] = jnp.ones((KPAD, IDIM), jnp.float32)
        accv_ref[...] = jnp.zeros_like(accv_ref)

    xo_refs = (xo1_ref, xo2_ref, xo3_ref, xo4_ref, xo5_ref, xo6_ref,
               xo7_ref, xo8_ref)
    for h, xo_ref in enumerate(xo_refs):
        x = x_ref[0, h * HBLK:(h + 1) * HBLK, :]
        s = jax.lax.dot_general(x, ea_ref[...], (((1,), (1,)), ((), ())),
                                preferred_element_type=jnp.float32)
        s = s + kb_ref[...]
        m = jnp.max(s, axis=1, keepdims=True)
        onehot = (s == m).astype(jnp.float32)
        p2 = jax.lax.dot(onehot, ew_ref[...],
                         preferred_element_type=jnp.float32)
        p = p2[:, 0:IDIM] / p2[:, IDIM:2 * IDIM] + b_ref[...]

        xo = xo_ref[0]
        t2 = (xo[:, 0:128] + xo[:, 128:256] + xo[:, 256:384]
              + xo[:, 384:512] + xo[:, 512:640])
        t = (t2[:, 0:IDIM] + t2[:, IDIM:2 * IDIM]) * (1.0 / TNUM)

        v = valid_ref[0, h * HBLK:(h + 1) * HBLK, :]
        accv_ref[...] += (CA * (p * p) - CB * (p * t) + t * t) * v

    @pl.when(i == nsteps - 1)
    def _fin():
        out_ref[...] = jnp.reshape(jnp.sum(accv_ref[...]), (1, 1))


def _run(xs_pad_in, xs_pad_out, ilens, embed_weight, W_inf, b_inf,
         interpret=False):
    B, T, _ = xs_pad_in.shape
    xso = xs_pad_out.reshape(B, T, TNUM * IDIM)
    valid = (jnp.arange(T, dtype=jnp.int32)[None, :, None]
             < ilens[:, None, None].astype(jnp.int32)).astype(jnp.float32)
    epad = jnp.zeros((KPAD, IDIM), jnp.float32).at[:K, :].set(embed_weight)
    kb = jnp.where(jnp.arange(KPAD)[None, :] < K, 0.0, -1e30
                   ).astype(jnp.float32)
    b2 = b_inf.reshape(1, IDIM)

    grid = (B,)
    out = pl.pallas_call(
        _vq_loss_kernel,
        grid=grid,
        in_specs=[
            pl.BlockSpec((1, T, IDIM), lambda i: (i, 0, 0)),
            *[pl.BlockSpec((1, HBLK, TNUM * IDIM),
                           (lambda q: (lambda i: (i, q, 0)))(q))
              for q in range(8)],
            pl.BlockSpec((1, T, 1), lambda i: (i, 0, 0)),
            pl.BlockSpec((KPAD, IDIM), lambda i: (0, 0)),
            pl.BlockSpec((IDIM, IDIM), lambda i: (0, 0)),
            pl.BlockSpec((1, IDIM), lambda i: (0, 0)),
            pl.BlockSpec((1, KPAD), lambda i: (0, 0)),
        ],
        out_specs=pl.BlockSpec((1, 1), lambda i: (0, 0)),
        out_shape=jax.ShapeDtypeStruct((1, 1), jnp.float32),
        scratch_shapes=[
            pltpu.VMEM((KPAD, IDIM), jnp.float32),
            pltpu.VMEM((KPAD, 2 * IDIM), jnp.float32),
            pltpu.VMEM((HBLK, IDIM), jnp.float32),
        ],
        interpret=interpret,
    )(xs_pad_in, *([xso] * 8), valid, epad, W_inf, b2, kb)
    return out.reshape(())


def kernel(xs_pad_in, xs_pad_out, ilens, ys_pad, embed_weight, W_inf, b_inf):
    return _run(xs_pad_in, xs_pad_out, ilens, embed_weight, W_inf, b_inf)
